# Initial kernel scaffold; baseline (speedup 1.0000x reference)
import jax
import jax.numpy as jnp
from jax.experimental import pallas as pl


def kernel(input_bits, state_memory, output_memory, conn_state, conn_out):
    raise NotImplementedError("write your pallas kernel here")


# speedup vs baseline: 318.0898x; 318.0898x over previous
import functools

import jax
import jax.numpy as jnp
from jax import lax
from jax.experimental import pallas as pl
from jax.experimental.pallas import tpu as pltpu
from jax.experimental.pallas import tpu_sc as plsc

NUM_HEADS = 64
K_BITS = 16
V_BITS = 16
NEURONS = 8
NBN_STATE = 12
NBN_OUT = 8
WIN = K_BITS + V_BITS
NWIN = 4096
CHUNK = 2048
GROUPS_PER_CHUNK = CHUNK // 16


def _body(bits_hbm, smem_hbm, omem_hbm, cst_hbm, cot_hbm, out_hbm,
          win_v, krev_v, head_v, wr_v, chain_v, sm_v, om_v, cst_v, cot_v,
          out_v, sem):
    cid = lax.axis_index("c")
    sid = lax.axis_index("s")

    @pl.when(jnp.logical_and(cid == 0, sid == 0))
    def _run():
        iota = lax.iota(jnp.int32, 16)
        base = iota * WIN

        maxq0 = jnp.full((16,), -1, jnp.int32)

        def make_group_body(chunk):
            def group_body(g, maxq):
                w0 = chunk * CHUNK + g * 16
                boff = base + (g * (16 * WIN))
                krev = plsc.load_gather(win_v, [boff])
                headv = jnp.zeros((16,), jnp.int32)
                for i in range(1, K_BITS):
                    b = plsc.load_gather(win_v, [boff + i])
                    krev = krev | (b << i)
                    if i >= 10:
                        headv = headv | (b << (15 - i))
                wrv = plsc.load_gather(win_v, [boff + K_BITS])
                for i in range(K_BITS + 1, WIN):
                    wrv = wrv | plsc.load_gather(win_v, [boff + i])
                krev_v[pl.ds(w0, 16)] = krev
                head_v[pl.ds(w0, 16)] = headv
                wr_v[pl.ds(w0, 16)] = wrv
                widx = iota + w0
                return jnp.maximum(maxq, jnp.where(wrv > 0, -1, widx))
            return group_body

        maxq = maxq0
        for chunk in range(NWIN // CHUNK):
            pltpu.sync_copy(bits_hbm.at[pl.ds(chunk * CHUNK * WIN, CHUNK * WIN)],
                            win_v)
            maxq = lax.fori_loop(0, GROUPS_PER_CHUNK, make_group_body(chunk),
                                 maxq)

        wq = jnp.max(maxq)
        wstar = jnp.where(wq < 0, NWIN - 1, wq).astype(jnp.int32)
        wsv = jnp.full((16,), wstar, jnp.int32)
        hstar = jnp.max(plsc.load_gather(head_v, [wsv]))
        krev_star = jnp.max(plsc.load_gather(krev_v, [wsv]))

        pltpu.sync_copy(smem_hbm.at[hstar], sm_v)
        pltpu.sync_copy(omem_hbm.at[hstar], om_v)
        pltpu.sync_copy(cst_hbm.at[hstar], cst_v)
        pltpu.sync_copy(cot_hbm.at[hstar], cot_v)

        def comp_body(g, cnt):
            off = g * 16
            kr = krev_v[pl.ds(off, 16)]
            hd = head_v[pl.ds(off, 16)]
            wr = wr_v[pl.ds(off, 16)]
            widx = iota + off
            m = jnp.logical_and(jnp.logical_and(hd == hstar, wr > 0),
                                widx < wstar)
            plsc.store_compressed(chain_v.at[pl.ds(cnt, 16)], kr, mask=m)
            return cnt + jnp.sum(m.astype(jnp.int32))

        cnt = lax.fori_loop(0, NWIN // 16, comp_body, jnp.int32(0))

        cs = [cst_v[pl.ds(16 * j, 16)] for j in range(NBN_STATE)]
        nid = (iota & 7) * (2 ** NBN_STATE)
        lane_lt8 = iota < 8

        def new_state_mask(inp_mask):
            inp_v = jnp.full((16,), inp_mask, jnp.int32)
            addr = (lax.shift_right_logical(inp_v, cs[0]) & 1)
            for j in range(1, NBN_STATE):
                addr = addr | ((lax.shift_right_logical(inp_v, cs[j]) & 1) << j)
            vals = plsc.load_gather(sm_v, [nid + addr])
            bits = jnp.logical_and(vals > 0.5, lane_lt8).astype(jnp.int32)
            return jnp.sum(bits << iota)

        def chain_body(t, smask):
            krev_t = chain_v[pl.ds(t, 16)][0]
            return new_state_mask(krev_t | (smask << K_BITS))

        smask = lax.fori_loop(0, cnt, chain_body, jnp.int32(0))

        s = new_state_mask(krev_star | (smask << K_BITS))
        sv = jnp.full((16,), s, jnp.int32)
        co0 = cot_v[pl.ds(0, 16)]
        addr2 = lax.shift_right_logical(sv, co0) & 1
        for j in range(1, NBN_OUT):
            coj = cot_v[pl.ds(16 * j, 16)]
            addr2 = addr2 | ((lax.shift_right_logical(sv, coj) & 1) << j)
        out_v[...] = plsc.load_gather(om_v, [iota * (2 ** NBN_OUT) + addr2])
        pltpu.sync_copy(out_v, out_hbm)


_sc_call = functools.partial(
    pl.kernel,
    out_type=jax.ShapeDtypeStruct((V_BITS,), jnp.float32),
    mesh=plsc.VectorSubcoreMesh(core_axis_name="c", subcore_axis_name="s"),
    scratch_types=[
        pltpu.VMEM((CHUNK * WIN,), jnp.int32),
        pltpu.VMEM((NWIN,), jnp.int32),
        pltpu.VMEM((NWIN,), jnp.int32),
        pltpu.VMEM((NWIN,), jnp.int32),
        pltpu.VMEM((NWIN + 16,), jnp.int32),
        pltpu.VMEM((NEURONS * 2 ** NBN_STATE,), jnp.float32),
        pltpu.VMEM((V_BITS * 2 ** NBN_OUT,), jnp.float32),
        pltpu.VMEM((NBN_STATE * 16,), jnp.int32),
        pltpu.VMEM((NBN_OUT * 16,), jnp.int32),
        pltpu.VMEM((V_BITS,), jnp.float32),
        pltpu.SemaphoreType.DMA,
    ],
    compiler_params=pltpu.CompilerParams(needs_layout_passes=False),
)(_body)


def kernel(input_bits, state_memory, output_memory, conn_state, conn_out):
    bits = input_bits.astype(jnp.int32)
    smem2 = state_memory.reshape(NUM_HEADS, NEURONS * 2 ** NBN_STATE)
    omem2 = output_memory.reshape(NUM_HEADS, V_BITS * 2 ** NBN_OUT)
    cst = jnp.transpose(conn_state.astype(jnp.int32), (0, 2, 1))
    cst = jnp.pad(cst, ((0, 0), (0, 0), (0, 16 - NEURONS)))
    cst = cst.reshape(NUM_HEADS, NBN_STATE * 16)
    cot = jnp.transpose(conn_out.astype(jnp.int32), (0, 2, 1))
    cot = cot.reshape(NUM_HEADS, NBN_OUT * 16)
    return _sc_call(bits, smem2, omem2, cst, cot)


# speedup vs baseline: 750.9406x; 2.3608x over previous
import functools

import jax
import jax.numpy as jnp
from jax import lax
from jax.experimental import pallas as pl
from jax.experimental.pallas import tpu as pltpu
from jax.experimental.pallas import tpu_sc as plsc

NUM_HEADS = 64
K_BITS = 16
V_BITS = 16
NEURONS = 8
NBN_STATE = 12
NBN_OUT = 8
WIN = K_BITS + V_BITS
NWIN = 4096
NSUB = 16
WPT = NWIN // NSUB
GPT = WPT // 16


def _body(bits_hbm, smem_hbm, omem_hbm, cst_hbm, cot_hbm, out_hbm,
          win_v, krev_l, head_l, wr_l, mq_l,
          krev_v, head_v, wr_v, mq_v, chain_v, sm_v, om_v, cst_v, cot_v,
          out_v, krev_s, head_s, wr_s, mq_s, sem):
    cid = lax.axis_index("c")
    sid = lax.axis_index("s")

    @pl.when(cid == 0)
    def _route():
        iota = lax.iota(jnp.int32, 16)
        base = iota * WIN
        w0_sub = sid * WPT

        pltpu.sync_copy(bits_hbm.at[pl.ds(w0_sub * WIN, WPT * WIN)], win_v)

        def group_body(g, maxq):
            boff = base + g * (16 * WIN)
            krev = plsc.load_gather(win_v, [boff])
            headv = jnp.zeros((16,), jnp.int32)
            for i in range(1, K_BITS):
                b = plsc.load_gather(win_v, [boff + i])
                krev = krev | (b << i)
                if i >= 10:
                    headv = headv | (b << (15 - i))
            wrv = plsc.load_gather(win_v, [boff + K_BITS])
            for i in range(K_BITS + 1, WIN):
                wrv = wrv | plsc.load_gather(win_v, [boff + i])
            off = g * 16
            krev_l[pl.ds(off, 16)] = krev
            head_l[pl.ds(off, 16)] = headv
            wr_l[pl.ds(off, 16)] = wrv
            widx = iota + (w0_sub + off)
            return jnp.maximum(maxq, jnp.where(wrv > 0, -1, widx))

        maxq = lax.fori_loop(0, GPT, group_body, jnp.full((16,), -1, jnp.int32))
        mq_l[...] = maxq
        pltpu.sync_copy(krev_l, krev_s.at[pl.ds(w0_sub, WPT)])
        pltpu.sync_copy(head_l, head_s.at[pl.ds(w0_sub, WPT)])
        pltpu.sync_copy(wr_l, wr_s.at[pl.ds(w0_sub, WPT)])
        pltpu.sync_copy(mq_l, mq_s.at[pl.ds(sid * 16, 16)])
        plsc.subcore_barrier()

    @pl.when(jnp.logical_and(cid == 0, sid == 0))
    def _tail():
        iota = lax.iota(jnp.int32, 16)

        pltpu.sync_copy(krev_s, krev_v)
        pltpu.sync_copy(head_s, head_v)
        pltpu.sync_copy(wr_s, wr_v)
        pltpu.sync_copy(mq_s, mq_v)

        def mq_body(i, maxq):
            return jnp.maximum(maxq, mq_v[pl.ds(i * 16, 16)])
        maxq = lax.fori_loop(0, NSUB, mq_body, jnp.full((16,), -1, jnp.int32))
        wq = jnp.max(maxq)
        wstar = jnp.where(wq < 0, NWIN - 1, wq).astype(jnp.int32)
        wsv = jnp.full((16,), wstar, jnp.int32)
        hstar = jnp.max(plsc.load_gather(head_v, [wsv]))
        krev_star = jnp.max(plsc.load_gather(krev_v, [wsv]))

        c1 = pltpu.async_copy(smem_hbm.at[hstar], sm_v, sem)
        c2 = pltpu.async_copy(omem_hbm.at[hstar], om_v, sem)
        c3 = pltpu.async_copy(cst_hbm.at[hstar], cst_v, sem)
        c4 = pltpu.async_copy(cot_hbm.at[hstar], cot_v, sem)

        def comp_body(g, cnt):
            off = g * 16
            kr = krev_v[pl.ds(off, 16)]
            hd = head_v[pl.ds(off, 16)]
            wr = wr_v[pl.ds(off, 16)]
            widx = iota + off
            m = jnp.logical_and(jnp.logical_and(hd == hstar, wr > 0),
                                widx < wstar)
            plsc.store_compressed(chain_v.at[pl.ds(cnt, 16)], kr, mask=m)
            return cnt + jnp.sum(m.astype(jnp.int32))

        cnt = lax.fori_loop(0, NWIN // 16, comp_body, jnp.int32(0))

        c1.wait()
        c2.wait()
        c3.wait()
        c4.wait()

        nid = iota & 7
        cs = [plsc.load_gather(cst_v, [nid * NBN_STATE + j])
              for j in range(NBN_STATE)]
        nid_off = nid * (2 ** NBN_STATE)
        lane_lt8 = iota < 8

        def new_state_mask(inp_mask):
            inp_v = jnp.full((16,), inp_mask, jnp.int32)
            addr = (lax.shift_right_logical(inp_v, cs[0]) & 1)
            for j in range(1, NBN_STATE):
                addr = addr | ((lax.shift_right_logical(inp_v, cs[j]) & 1) << j)
            vals = plsc.load_gather(sm_v, [nid_off + addr])
            bits = jnp.logical_and(vals > 0.5, lane_lt8).astype(jnp.int32)
            return jnp.sum(bits << iota)

        def chain_body(t, smask):
            krev_t = chain_v[pl.ds(t, 16)][0]
            return new_state_mask(krev_t | (smask << K_BITS))

        smask = lax.fori_loop(0, cnt, chain_body, jnp.int32(0))

        s = new_state_mask(krev_star | (smask << K_BITS))
        sv = jnp.full((16,), s, jnp.int32)
        co = [plsc.load_gather(cot_v, [iota * NBN_OUT + j])
              for j in range(NBN_OUT)]
        addr2 = lax.shift_right_logical(sv, co[0]) & 1
        for j in range(1, NBN_OUT):
            addr2 = addr2 | ((lax.shift_right_logical(sv, co[j]) & 1) << j)
        out_v[...] = plsc.load_gather(om_v, [iota * (2 ** NBN_OUT) + addr2])
        pltpu.sync_copy(out_v, out_hbm)


_sc_call = functools.partial(
    pl.kernel,
    out_type=jax.ShapeDtypeStruct((V_BITS,), jnp.float32),
    mesh=plsc.VectorSubcoreMesh(core_axis_name="c", subcore_axis_name="s"),
    scratch_types=[
        pltpu.VMEM((WPT * WIN,), jnp.int32),
        pltpu.VMEM((WPT,), jnp.int32),
        pltpu.VMEM((WPT,), jnp.int32),
        pltpu.VMEM((WPT,), jnp.int32),
        pltpu.VMEM((16,), jnp.int32),
        pltpu.VMEM((NWIN,), jnp.int32),
        pltpu.VMEM((NWIN,), jnp.int32),
        pltpu.VMEM((NWIN,), jnp.int32),
        pltpu.VMEM((NSUB * 16,), jnp.int32),
        pltpu.VMEM((NWIN + 16,), jnp.int32),
        pltpu.VMEM((NEURONS * 2 ** NBN_STATE,), jnp.float32),
        pltpu.VMEM((V_BITS * 2 ** NBN_OUT,), jnp.float32),
        pltpu.VMEM((NEURONS * NBN_STATE,), jnp.int32),
        pltpu.VMEM((V_BITS * NBN_OUT,), jnp.int32),
        pltpu.VMEM((V_BITS,), jnp.float32),
        pltpu.VMEM_SHARED((NWIN,), jnp.int32),
        pltpu.VMEM_SHARED((NWIN,), jnp.int32),
        pltpu.VMEM_SHARED((NWIN,), jnp.int32),
        pltpu.VMEM_SHARED((NSUB * 16,), jnp.int32),
        pltpu.SemaphoreType.DMA,
    ],
    compiler_params=pltpu.CompilerParams(needs_layout_passes=False),
)(_body)


def kernel(input_bits, state_memory, output_memory, conn_state, conn_out):
    bits = input_bits.astype(jnp.int32)
    smem2 = state_memory.reshape(NUM_HEADS, NEURONS * 2 ** NBN_STATE)
    omem2 = output_memory.reshape(NUM_HEADS, V_BITS * 2 ** NBN_OUT)
    cst = conn_state.astype(jnp.int32).reshape(NUM_HEADS, NEURONS * NBN_STATE)
    cot = conn_out.astype(jnp.int32).reshape(NUM_HEADS, V_BITS * NBN_OUT)
    return _sc_call(bits, smem2, omem2, cst, cot)


# speedup vs baseline: 1005.3296x; 1.3388x over previous
import functools

import jax
import jax.numpy as jnp
from jax import lax
from jax.experimental import pallas as pl
from jax.experimental.pallas import tpu as pltpu
from jax.experimental.pallas import tpu_sc as plsc

NUM_HEADS = 64
K_BITS = 16
V_BITS = 16
NEURONS = 8
NBN_STATE = 12
NBN_OUT = 8
WIN = K_BITS + V_BITS
NWIN = 4096
NSUB = 16
WPT = NWIN // NSUB
GPT = WPT // 16


def _body(bits_hbm, smem_hbm, omem_hbm, cst_hbm, cot_hbm, out_hbm,
          win_v, krev_l, head_l, wr_l, mq_l,
          krev_v, head_v, wr_v, mq_v, chain_v, sm_v, om_v, cst_v, cot_v,
          out_v, krev_s, head_s, wr_s, mq_s, sem):
    cid = lax.axis_index("c")
    sid = lax.axis_index("s")

    @pl.when(cid == 0)
    def _route():
        iota = lax.iota(jnp.int32, 16)
        base = iota * WIN
        w0_sub = sid * WPT

        pltpu.sync_copy(bits_hbm.at[pl.ds(w0_sub * WIN, WPT * WIN)], win_v)

        def group_body(g, maxq):
            boff = base + g * (16 * WIN)
            krev = plsc.load_gather(win_v, [boff])
            headv = jnp.zeros((16,), jnp.int32)
            for i in range(1, K_BITS):
                b = plsc.load_gather(win_v, [boff + i])
                krev = krev | (b << i)
                if i >= 10:
                    headv = headv | (b << (15 - i))
            wrv = plsc.load_gather(win_v, [boff + K_BITS])
            for i in range(K_BITS + 1, WIN):
                wrv = wrv | plsc.load_gather(win_v, [boff + i])
            off = g * 16
            krev_l[pl.ds(off, 16)] = krev
            head_l[pl.ds(off, 16)] = headv
            wr_l[pl.ds(off, 16)] = wrv
            widx = iota + (w0_sub + off)
            return jnp.maximum(maxq, jnp.where(wrv > 0, -1, widx))

        maxq = lax.fori_loop(0, GPT, group_body, jnp.full((16,), -1, jnp.int32))
        mq_l[...] = maxq
        pltpu.sync_copy(krev_l, krev_s.at[pl.ds(w0_sub, WPT)])
        pltpu.sync_copy(head_l, head_s.at[pl.ds(w0_sub, WPT)])
        pltpu.sync_copy(wr_l, wr_s.at[pl.ds(w0_sub, WPT)])
        pltpu.sync_copy(mq_l, mq_s.at[pl.ds(sid * 16, 16)])
        plsc.subcore_barrier()

    @pl.when(jnp.logical_and(cid == 0, sid == 0))
    def _tail():
        iota = lax.iota(jnp.int32, 16)

        pltpu.sync_copy(krev_s, krev_v)
        pltpu.sync_copy(head_s, head_v)
        pltpu.sync_copy(wr_s, wr_v)
        pltpu.sync_copy(mq_s, mq_v)

        def mq_body(i, maxq):
            return jnp.maximum(maxq, mq_v[pl.ds(i * 16, 16)])
        maxq = lax.fori_loop(0, NSUB, mq_body, jnp.full((16,), -1, jnp.int32))
        wq = jnp.max(maxq)
        wstar = jnp.where(wq < 0, NWIN - 1, wq).astype(jnp.int32)
        wsv = jnp.full((16,), wstar, jnp.int32)
        hstar = jnp.max(plsc.load_gather(head_v, [wsv]))
        krev_star = jnp.max(plsc.load_gather(krev_v, [wsv]))

        c1 = pltpu.async_copy(smem_hbm.at[hstar], sm_v, sem)
        c2 = pltpu.async_copy(omem_hbm.at[hstar], om_v, sem)
        c3 = pltpu.async_copy(cst_hbm.at[hstar], cst_v, sem)
        c4 = pltpu.async_copy(cot_hbm.at[hstar], cot_v, sem)

        def comp_body(g, cnt):
            off = g * 16
            kr = krev_v[pl.ds(off, 16)]
            hd = head_v[pl.ds(off, 16)]
            wr = wr_v[pl.ds(off, 16)]
            widx = iota + off
            m = jnp.logical_and(jnp.logical_and(hd == hstar, wr > 0),
                                widx < wstar)
            plsc.store_compressed(chain_v.at[pl.ds(cnt, 16)], kr, mask=m)
            return cnt + jnp.sum(m.astype(jnp.int32))

        cnt = lax.fori_loop(0, NWIN // 16, comp_body, jnp.int32(0))

        c1.wait()
        c2.wait()
        c3.wait()
        c4.wait()

        nid = iota & 7
        cs = [plsc.load_gather(cst_v, [nid, jnp.full((16,), j, jnp.int32)])
              for j in range(NBN_STATE)]
        lane_lt8 = iota < 8

        def new_state_mask(inp_mask):
            inp_v = jnp.full((16,), inp_mask, jnp.int32)
            addr = (lax.shift_right_logical(inp_v, cs[0]) & 1)
            for j in range(1, NBN_STATE):
                addr = addr | ((lax.shift_right_logical(inp_v, cs[j]) & 1) << j)
            vals = plsc.load_gather(sm_v, [nid, addr])
            bits = jnp.logical_and(vals > 0.5, lane_lt8).astype(jnp.int32)
            return jnp.sum(bits << iota)

        def chain_body(t, smask):
            krev_t = chain_v[pl.ds(t, 16)][0]
            return new_state_mask(krev_t | (smask << K_BITS))

        smask = lax.fori_loop(0, cnt, chain_body, jnp.int32(0))

        s = new_state_mask(krev_star | (smask << K_BITS))
        sv = jnp.full((16,), s, jnp.int32)
        co = [plsc.load_gather(cot_v, [iota, jnp.full((16,), j, jnp.int32)])
              for j in range(NBN_OUT)]
        addr2 = lax.shift_right_logical(sv, co[0]) & 1
        for j in range(1, NBN_OUT):
            addr2 = addr2 | ((lax.shift_right_logical(sv, co[j]) & 1) << j)
        out_v[...] = plsc.load_gather(om_v, [iota, addr2])
        pltpu.sync_copy(out_v, out_hbm)


_sc_call = functools.partial(
    pl.kernel,
    out_type=jax.ShapeDtypeStruct((V_BITS,), jnp.float32),
    mesh=plsc.VectorSubcoreMesh(core_axis_name="c", subcore_axis_name="s"),
    scratch_types=[
        pltpu.VMEM((WPT * WIN,), jnp.int32),
        pltpu.VMEM((WPT,), jnp.int32),
        pltpu.VMEM((WPT,), jnp.int32),
        pltpu.VMEM((WPT,), jnp.int32),
        pltpu.VMEM((16,), jnp.int32),
        pltpu.VMEM((NWIN,), jnp.int32),
        pltpu.VMEM((NWIN,), jnp.int32),
        pltpu.VMEM((NWIN,), jnp.int32),
        pltpu.VMEM((NSUB * 16,), jnp.int32),
        pltpu.VMEM((NWIN + 16,), jnp.int32),
        pltpu.VMEM((NEURONS, 2 ** NBN_STATE), jnp.float32),
        pltpu.VMEM((V_BITS, 2 ** NBN_OUT), jnp.float32),
        pltpu.VMEM((NEURONS, NBN_STATE), jnp.int32),
        pltpu.VMEM((V_BITS, NBN_OUT), jnp.int32),
        pltpu.VMEM((V_BITS,), jnp.float32),
        pltpu.VMEM_SHARED((NWIN,), jnp.int32),
        pltpu.VMEM_SHARED((NWIN,), jnp.int32),
        pltpu.VMEM_SHARED((NWIN,), jnp.int32),
        pltpu.VMEM_SHARED((NSUB * 16,), jnp.int32),
        pltpu.SemaphoreType.DMA,
    ],
    compiler_params=pltpu.CompilerParams(needs_layout_passes=False),
)(_body)


def kernel(input_bits, state_memory, output_memory, conn_state, conn_out):
    return _sc_call(input_bits.astype(jnp.int32), state_memory, output_memory,
                    conn_state.astype(jnp.int32), conn_out.astype(jnp.int32))


# speedup vs baseline: 1023.6919x; 1.0183x over previous
import functools

import jax
import jax.numpy as jnp
from jax import lax
from jax.experimental import pallas as pl
from jax.experimental.pallas import tpu as pltpu
from jax.experimental.pallas import tpu_sc as plsc

NUM_HEADS = 64
K_BITS = 16
V_BITS = 16
NEURONS = 8
NBN_STATE = 12
NBN_OUT = 8
WIN = K_BITS + V_BITS
NWIN = 4096
NSUB = 16
WPT = NWIN // NSUB
GPT = WPT // 16


def _body(bits_hbm, smem_hbm, omem_hbm, cst_hbm, cot_hbm, out_hbm,
          win_v, krev_l, head_l, wr_l, stage_l, seg_l, mq_v, hk_v,
          seg_v, cnt_v, sm_v, om_v, cst_v, cot_v, out_v,
          mq_s, hk_s, seg_s, cnt_s, sem):
    cid = lax.axis_index("c")
    sid = lax.axis_index("s")

    @pl.when(cid == 0)
    def _route():
        iota = lax.iota(jnp.int32, 16)
        base = iota * WIN
        w0_sub = sid * WPT

        pltpu.sync_copy(bits_hbm.at[pl.ds(w0_sub * WIN, WPT * WIN)], win_v)

        def group_body(g, maxq):
            boff = base + g * (16 * WIN)
            krev = plsc.load_gather(win_v, [boff])
            headv = jnp.zeros((16,), jnp.int32)
            for i in range(1, K_BITS):
                b = plsc.load_gather(win_v, [boff + i])
                krev = krev | (b << i)
                if i >= 10:
                    headv = headv | (b << (15 - i))
            wrv = plsc.load_gather(win_v, [boff + K_BITS])
            for i in range(K_BITS + 1, WIN):
                wrv = wrv | plsc.load_gather(win_v, [boff + i])
            off = g * 16
            krev_l[pl.ds(off, 16)] = krev
            head_l[pl.ds(off, 16)] = headv
            wr_l[pl.ds(off, 16)] = wrv
            widx = iota + (w0_sub + off)
            return jnp.maximum(maxq, jnp.where(wrv > 0, -1, widx))

        maxq = lax.fori_loop(0, GPT, group_body, jnp.full((16,), -1, jnp.int32))
        stage_l[...] = maxq
        pltpu.sync_copy(stage_l, mq_s.at[pl.ds(sid * 16, 16)])
        plsc.subcore_barrier()

        pltpu.sync_copy(mq_s, mq_v)

        def mq_body(i, mq):
            return jnp.maximum(mq, mq_v[pl.ds(i * 16, 16)])
        mq = lax.fori_loop(0, NSUB, mq_body, jnp.full((16,), -1, jnp.int32))
        wq = jnp.max(mq)
        wstar = jnp.where(wq < 0, NWIN - 1, wq).astype(jnp.int32)

        @pl.when(wstar // WPT == sid)
        def _publish_hk():
            lidx = jnp.full((16,), wstar - w0_sub, jnp.int32)
            stage_l[...] = plsc.load_gather(head_l, [lidx])
            pltpu.sync_copy(stage_l, hk_s.at[pl.ds(0, 16)])
            stage_l[...] = plsc.load_gather(krev_l, [lidx])
            pltpu.sync_copy(stage_l, hk_s.at[pl.ds(16, 16)])
        plsc.subcore_barrier()

        pltpu.sync_copy(hk_s, hk_v)
        hstar = hk_v[pl.ds(0, 16)][0]

        @pl.when(sid == 0)
        def _fire_dma():
            pltpu.async_copy(smem_hbm.at[hstar], sm_v, sem)
            pltpu.async_copy(omem_hbm.at[hstar], om_v, sem)
            pltpu.async_copy(cst_hbm.at[hstar], cst_v, sem)
            pltpu.async_copy(cot_hbm.at[hstar], cot_v, sem)

        def comp_body(g, cnt):
            off = g * 16
            kr = krev_l[pl.ds(off, 16)]
            hd = head_l[pl.ds(off, 16)]
            wr = wr_l[pl.ds(off, 16)]
            widx = iota + (w0_sub + off)
            m = jnp.logical_and(jnp.logical_and(hd == hstar, wr > 0),
                                widx < wstar)
            plsc.store_compressed(seg_l.at[pl.ds(cnt, 16)], kr, mask=m)
            return cnt + jnp.sum(m.astype(jnp.int32))

        cnt = lax.fori_loop(0, GPT, comp_body, jnp.int32(0))
        stage_l[...] = jnp.full((16,), cnt, jnp.int32)
        pltpu.sync_copy(stage_l, cnt_s.at[pl.ds(sid * 16, 16)])
        pltpu.sync_copy(seg_l.at[pl.ds(0, WPT)], seg_s.at[pl.ds(sid * WPT, WPT)])
        plsc.subcore_barrier()

    @pl.when(jnp.logical_and(cid == 0, sid == 0))
    def _tail():
        iota = lax.iota(jnp.int32, 16)
        pltpu.sync_copy(seg_s, seg_v.at[pl.ds(0, NWIN)])
        pltpu.sync_copy(cnt_s, cnt_v)
        krev_star = hk_v[pl.ds(16, 16)][0]

        pltpu.make_async_copy(smem_hbm.at[0], sm_v, sem).wait()
        pltpu.make_async_copy(omem_hbm.at[0], om_v, sem).wait()
        pltpu.make_async_copy(cst_hbm.at[0], cst_v, sem).wait()
        pltpu.make_async_copy(cot_hbm.at[0], cot_v, sem).wait()

        nid = iota & 7
        cs = [plsc.load_gather(cst_v, [nid, jnp.full((16,), j, jnp.int32)])
              for j in range(NBN_STATE)]
        lane_lt8 = iota < 8

        def new_state_mask(inp_mask):
            inp_v = jnp.full((16,), inp_mask, jnp.int32)
            addr = (lax.shift_right_logical(inp_v, cs[0]) & 1)
            for j in range(1, NBN_STATE):
                addr = addr | ((lax.shift_right_logical(inp_v, cs[j]) & 1) << j)
            vals = plsc.load_gather(sm_v, [nid, addr])
            bits = jnp.logical_and(vals > 0.5, lane_lt8).astype(jnp.int32)
            return jnp.sum(bits << iota)

        def seg_body(i, smask0):
            cnt_i = cnt_v[pl.ds(i * 16, 16)][0]
            off = i * WPT

            def chain_body(t, smask):
                krev_t = seg_v[pl.ds(off + t, 16)][0]
                return new_state_mask(krev_t | (smask << K_BITS))
            return lax.fori_loop(0, cnt_i, chain_body, smask0)

        smask = lax.fori_loop(0, NSUB, seg_body, jnp.int32(0))

        s = new_state_mask(krev_star | (smask << K_BITS))
        sv = jnp.full((16,), s, jnp.int32)
        co = [plsc.load_gather(cot_v, [iota, jnp.full((16,), j, jnp.int32)])
              for j in range(NBN_OUT)]
        addr2 = lax.shift_right_logical(sv, co[0]) & 1
        for j in range(1, NBN_OUT):
            addr2 = addr2 | ((lax.shift_right_logical(sv, co[j]) & 1) << j)
        out_v[...] = plsc.load_gather(om_v, [iota, addr2])
        pltpu.sync_copy(out_v, out_hbm)


_sc_call = functools.partial(
    pl.kernel,
    out_type=jax.ShapeDtypeStruct((V_BITS,), jnp.float32),
    mesh=plsc.VectorSubcoreMesh(core_axis_name="c", subcore_axis_name="s"),
    scratch_types=[
        pltpu.VMEM((WPT * WIN,), jnp.int32),
        pltpu.VMEM((WPT,), jnp.int32),
        pltpu.VMEM((WPT,), jnp.int32),
        pltpu.VMEM((WPT,), jnp.int32),
        pltpu.VMEM((16,), jnp.int32),
        pltpu.VMEM((WPT + 16,), jnp.int32),
        pltpu.VMEM((NSUB * 16,), jnp.int32),
        pltpu.VMEM((32,), jnp.int32),
        pltpu.VMEM((NWIN + 16,), jnp.int32),
        pltpu.VMEM((NSUB * 16,), jnp.int32),
        pltpu.VMEM((NEURONS, 2 ** NBN_STATE), jnp.float32),
        pltpu.VMEM((V_BITS, 2 ** NBN_OUT), jnp.float32),
        pltpu.VMEM((NEURONS, NBN_STATE), jnp.int32),
        pltpu.VMEM((V_BITS, NBN_OUT), jnp.int32),
        pltpu.VMEM((V_BITS,), jnp.float32),
        pltpu.VMEM_SHARED((NSUB * 16,), jnp.int32),
        pltpu.VMEM_SHARED((32,), jnp.int32),
        pltpu.VMEM_SHARED((NWIN,), jnp.int32),
        pltpu.VMEM_SHARED((NSUB * 16,), jnp.int32),
        pltpu.SemaphoreType.DMA,
    ],
    compiler_params=pltpu.CompilerParams(needs_layout_passes=False),
)(_body)


def kernel(input_bits, state_memory, output_memory, conn_state, conn_out):
    return _sc_call(input_bits.astype(jnp.int32), state_memory, output_memory,
                    conn_state.astype(jnp.int32), conn_out.astype(jnp.int32))
